# Initial kernel scaffold; baseline (speedup 1.0000x reference)
import jax
import jax.numpy as jnp
from jax.experimental import pallas as pl


def kernel(x, edge_index, edge_attr, W1, b1, W2, b2, cw1, cb1, cw2, cb2):
    raise NotImplementedError("write your pallas kernel here")


# speedup vs baseline: 47.5641x; 47.5641x over previous
import dataclasses
import functools

import jax
import jax.numpy as jnp
from jax import lax
from jax.experimental import pallas as pl
from jax.experimental.pallas import tpu as pltpu
from jax.experimental.pallas import tpu_sc as plsc

N = 10000
E = 160000
NPAD = 10240
NT = 16
CH = NPAD // NT
ET = E // NT
L = 16


def _bits_u32(x):
    return lax.bitcast_convert_type(x, jnp.uint32)


def _rsqrt_newton(x):
    i = lax.bitcast_convert_type(x, jnp.int32)
    y = lax.bitcast_convert_type(jnp.int32(0x5F3759DF) - (i >> 1), jnp.float32)
    for _ in range(3):
        y = y * (1.5 - 0.5 * x * y * y)
    return y


def _sc_body(src_hbm, dst_hbm, ew_hbm, xw_hbm, par_hbm,
             h1s_hbm, h2s_hbm,
             srcv, dstv, ewv, nrmv, msgs,
             nodea, nodeb, h1f, h2f,
             dissl, asl, ssl, h1sl, h2sl, zb, o1, o2,
             kc, vc, ks, vs, dg, pls, gp,
             hist, Sv, tot, pmv, gv,
             scrd, scrk, scrv, parv,
             dega, acc1, acc2, disS, sS, h1S, h2S,
             skA, svA, skB, svB, Gs):
    cid = lax.axis_index("c")
    tid = lax.axis_index("s")

    @pl.when(cid == 0)
    def _work():
        ii = lax.iota(jnp.int32, L)
        nsl = pl.ds(tid * CH, CH)
        esl = pl.ds(tid * ET, ET)

        pltpu.sync_copy(src_hbm.at[esl], srcv)
        pltpu.sync_copy(dst_hbm.at[esl], dstv)
        pltpu.sync_copy(ew_hbm.at[esl], ewv)
        pltpu.sync_copy(xw_hbm, nodea)
        pltpu.sync_copy(par_hbm, parv)

        @pl.loop(0, CH // L)
        def _z(j):
            zb[pl.ds(j * L, L)] = jnp.zeros((L,), jnp.float32)

        pltpu.sync_copy(zb, dega.at[nsl])
        pltpu.sync_copy(zb, acc1.at[nsl])
        pltpu.sync_copy(zb, acc2.at[nsl])
        plsc.subcore_barrier()

        pltpu.sync_copy(ewv, dega.at[dstv], add=True)
        plsc.subcore_barrier()

        pltpu.sync_copy(dega.at[nsl], asl)

        @pl.loop(0, CH // L)
        def _dis(j):
            sl = pl.ds(j * L, L)
            dissl[sl] = _rsqrt_newton(asl[sl] + 1.0)

        pltpu.sync_copy(dissl, disS.at[nsl])
        plsc.subcore_barrier()
        pltpu.sync_copy(disS, nodeb)

        z16 = jnp.zeros((L,), jnp.int32)
        w2b = plsc.load_gather(parv, [z16 + 2])
        b1b = plsc.load_gather(parv, [z16 + 3])
        b2b = plsc.load_gather(parv, [z16 + 4])

        @pl.loop(0, ET // L)
        def _m1(j):
            sl = pl.ds(j * L, L)
            s_i = srcv[sl]
            d_i = dstv[sl]
            w = ewv[sl]
            nr = plsc.load_gather(nodeb, [s_i]) * w * plsc.load_gather(nodeb, [d_i])
            nrmv[sl] = nr
            msgs[sl] = nr * plsc.load_gather(nodea, [s_i])

        pltpu.sync_copy(msgs, acc1.at[dstv], add=True)
        plsc.subcore_barrier()

        pltpu.sync_copy(acc1.at[nsl], asl)

        @pl.loop(0, CH // L)
        def _h1(j):
            sl = pl.ds(j * L, L)
            di = dissl[sl]
            xwsl = nodea[pl.ds(tid * CH + j * L, L)]
            h1 = asl[sl] + di * di * xwsl + b1b
            h1sl[sl] = h1
            ssl[sl] = h1 * w2b

        pltpu.sync_copy(h1sl, h1S.at[nsl])
        pltpu.sync_copy(ssl, sS.at[nsl])
        plsc.subcore_barrier()
        pltpu.sync_copy(sS, nodea)
        pltpu.sync_copy(h1S, h1f)

        @pl.loop(0, ET // L)
        def _m2(j):
            sl = pl.ds(j * L, L)
            msgs[sl] = nrmv[sl] * plsc.load_gather(nodea, [srcv[sl]])

        pltpu.sync_copy(msgs, acc2.at[dstv], add=True)
        plsc.subcore_barrier()

        pltpu.sync_copy(acc2.at[nsl], asl)

        @pl.loop(0, CH // L)
        def _h2(j):
            sl = pl.ds(j * L, L)
            di = dissl[sl]
            h2 = asl[sl] + di * di * ssl[sl] + b2b
            h2sl[sl] = h2
            gi = tid * CH + j * L + ii
            u = _bits_u32(h2)
            neg = lax.bitcast_convert_type(h2, jnp.int32) < 0
            msk = jnp.where(neg, jnp.uint32(0xFFFFFFFF), jnp.uint32(0x80000000))
            key = (u ^ msk) ^ jnp.uint32(0xFFFFFFFF)
            key = jnp.where(gi >= N, jnp.uint32(0xFFFFFFFF), key)
            kc[sl] = key
            vc[sl] = gi

        pltpu.sync_copy(h2sl, h2S.at[nsl])
        pltpu.sync_copy(kc, skA.at[nsl])
        pltpu.sync_copy(vc, svA.at[nsl])
        plsc.subcore_barrier()
        pltpu.sync_copy(h2S, h2f)

        bufs = [(skA, svA), (skB, svB)]
        for p in range(4):
            sk_src, sv_src = bufs[p % 2]
            sk_dst, sv_dst = bufs[(p + 1) % 2]
            if p > 0:
                pltpu.sync_copy(sk_src.at[nsl], kc)
                pltpu.sync_copy(sv_src.at[nsl], vc)
            for q in range(256 // L):
                hist[pl.ds(q * L, L)] = jnp.zeros((L,), jnp.int32)

            shift = jnp.uint32(8 * p)

            @pl.loop(0, CH // L)
            def _local(j, shift=shift):
                sl = pl.ds(j * L, L)
                k = kc[sl]
                v = vc[sl]
                d = (k >> shift) & jnp.uint32(0xFF)
                packed = (d << jnp.uint32(4)) | ii.astype(jnp.uint32)
                sp, ln = plsc.sort_key_val(packed, ii)
                ds_ = (sp >> jnp.uint32(4)).astype(jnp.int32)
                scrd[...] = ds_
                prev = plsc.load_gather(scrd, [jnp.maximum(ii - 1, 0)])
                nxt = plsc.load_gather(scrd, [jnp.minimum(ii + 1, L - 1)])
                b = (ii == 0) | (ds_ != prev)
                rend = (ii == L - 1) | (ds_ != nxt)
                rstart = plsc.cummax(jnp.where(b, ii, 0))
                r = ii - rstart
                pfx = plsc.load_gather(hist, [ds_])
                pos = pfx + r
                plsc.store_scatter(hist, [ds_], pos + 1, mask=rend)
                scrk[...] = lax.bitcast_convert_type(k, jnp.int32)
                scrv[...] = v
                ks[sl] = lax.bitcast_convert_type(
                    plsc.load_gather(scrk, [ln]), jnp.uint32)
                vs[sl] = plsc.load_gather(scrv, [ln])
                dg[sl] = ds_
                pls[sl] = pos

            pltpu.sync_copy(hist, Gs.at[pl.ds(tid * 256, 256)])
            plsc.subcore_barrier()
            pltpu.sync_copy(Gs, gv)

            for q in range(256 // L):
                dsl = pl.ds(q * L, L)

                def _sum_rows(t, acc, dsl=dsl):
                    return acc + gv[pl.ds(t * 256 + q * L, L)]

                tot[dsl] = lax.fori_loop(0, NT, _sum_rows, jnp.zeros((L,), jnp.int32))
                pmv[dsl] = lax.fori_loop(0, tid, _sum_rows, jnp.zeros((L,), jnp.int32))

            carry = jnp.int32(0)
            for q in range(256 // L):
                dsl = pl.ds(q * L, L)
                ch = tot[dsl]
                inc = plsc.cumsum(ch)
                Sv[dsl] = (inc - ch) + carry + pmv[dsl]
                carry = carry + jnp.sum(ch)

            for q in range(CH // L):
                sl = pl.ds(q * L, L)
                gp[sl] = plsc.load_gather(Sv, [dg[sl]]) + pls[sl]

            pltpu.sync_copy(ks, sk_dst.at[gp])
            pltpu.sync_copy(vs, sv_dst.at[gp])
            plsc.subcore_barrier()

        pltpu.sync_copy(svA.at[nsl], vc)

        @pl.loop(0, CH // L)
        def _out(j):
            sl = pl.ds(j * L, L)
            idx = vc[sl]
            o1[sl] = plsc.load_gather(h1f, [idx])
            o2[sl] = plsc.load_gather(h2f, [idx])

        pltpu.sync_copy(o1, h1s_hbm.at[nsl])
        pltpu.sync_copy(o2, h2s_hbm.at[nsl])


def _sc_main(src, dst, ew, xwp, par):
    f32 = jnp.float32
    i32 = jnp.int32
    u32 = jnp.uint32
    mesh = plsc.VectorSubcoreMesh(core_axis_name="c", subcore_axis_name="s")
    cp = pltpu.CompilerParams()
    if "needs_layout_passes" in pltpu.CompilerParams.__dataclass_fields__:
        cp = dataclasses.replace(cp, needs_layout_passes=False)
    kern = pl.kernel(
        _sc_body,
        out_type=(jax.ShapeDtypeStruct((NPAD,), f32),
                  jax.ShapeDtypeStruct((NPAD,), f32)),
        mesh=mesh,
        compiler_params=cp,
        scratch_types=[
            pltpu.VMEM((ET,), i32), pltpu.VMEM((ET,), i32),
            pltpu.VMEM((ET,), f32), pltpu.VMEM((ET,), f32),
            pltpu.VMEM((ET,), f32),
            pltpu.VMEM((NPAD,), f32), pltpu.VMEM((NPAD,), f32),
            pltpu.VMEM((NPAD,), f32), pltpu.VMEM((NPAD,), f32),
            pltpu.VMEM((CH,), f32), pltpu.VMEM((CH,), f32),
            pltpu.VMEM((CH,), f32), pltpu.VMEM((CH,), f32),
            pltpu.VMEM((CH,), f32), pltpu.VMEM((CH,), f32),
            pltpu.VMEM((CH,), f32), pltpu.VMEM((CH,), f32),
            pltpu.VMEM((CH,), u32), pltpu.VMEM((CH,), i32),
            pltpu.VMEM((CH,), u32), pltpu.VMEM((CH,), i32),
            pltpu.VMEM((CH,), i32), pltpu.VMEM((CH,), i32),
            pltpu.VMEM((CH,), i32),
            pltpu.VMEM((256,), i32), pltpu.VMEM((256,), i32),
            pltpu.VMEM((256,), i32), pltpu.VMEM((256,), i32),
            pltpu.VMEM((NT * 256,), i32),
            pltpu.VMEM((L,), i32), pltpu.VMEM((L,), i32),
            pltpu.VMEM((L,), i32), pltpu.VMEM((L,), f32),
            pltpu.VMEM_SHARED((NPAD,), f32), pltpu.VMEM_SHARED((NPAD,), f32),
            pltpu.VMEM_SHARED((NPAD,), f32), pltpu.VMEM_SHARED((NPAD,), f32),
            pltpu.VMEM_SHARED((NPAD,), f32), pltpu.VMEM_SHARED((NPAD,), f32),
            pltpu.VMEM_SHARED((NPAD,), f32),
            pltpu.VMEM_SHARED((NPAD,), u32), pltpu.VMEM_SHARED((NPAD,), i32),
            pltpu.VMEM_SHARED((NPAD,), u32), pltpu.VMEM_SHARED((NPAD,), i32),
            pltpu.VMEM_SHARED((NT * 256,), i32),
        ],
    )
    return kern(src, dst, ew, xwp, par)


def _tc_matvec(x, W1):
    def body(x_ref, w_ref, o_ref):
        o_ref[...] = jnp.dot(x_ref[...], w_ref[...],
                             preferred_element_type=jnp.float32)

    return pl.pallas_call(
        body,
        out_shape=jax.ShapeDtypeStruct((x.shape[0], 1), jnp.float32),
    )(x, W1)


def _tc_convs(h1s, h2s, cw1, cb1, cw2, cb2):
    def body(h1_ref, h2_ref, w1_ref, b1_ref, w2_ref, b2_ref, o_ref):
        hs = (h1_ref[...], h2_ref[...])
        L1 = N - 2
        pooled = []
        for o in range(3):
            acc = jnp.full((1, L1), b1_ref[o], jnp.float32)
            for c in range(2):
                for t in range(3):
                    acc = acc + w1_ref[o, c, t] * lax.slice(hs[c], (0, t), (1, t + L1))
            m = jnp.maximum(jnp.maximum(lax.slice(acc, (0, 0), (1, L1 - 2)),
                                        lax.slice(acc, (0, 1), (1, L1 - 1))),
                            lax.slice(acc, (0, 2), (1, L1)))
            pooled.append(m)
        L2 = L1 - 4
        acc2 = jnp.full((1, L2), b2_ref[0], jnp.float32)
        for c in range(3):
            for t in range(3):
                acc2 = acc2 + w2_ref[0, c, t] * lax.slice(pooled[c], (0, t), (1, t + L2))
        y = jnp.maximum(jnp.maximum(lax.slice(acc2, (0, 0), (1, L2 - 2)),
                                    lax.slice(acc2, (0, 1), (1, L2 - 1))),
                        lax.slice(acc2, (0, 2), (1, L2)))
        o_ref[...] = y.reshape(1, 1, L2 - 2)

    return pl.pallas_call(
        body,
        out_shape=jax.ShapeDtypeStruct((1, 1, N - 8), jnp.float32),
        in_specs=[
            pl.BlockSpec(memory_space=pltpu.VMEM),
            pl.BlockSpec(memory_space=pltpu.VMEM),
            pl.BlockSpec(memory_space=pltpu.SMEM),
            pl.BlockSpec(memory_space=pltpu.SMEM),
            pl.BlockSpec(memory_space=pltpu.SMEM),
            pl.BlockSpec(memory_space=pltpu.SMEM),
        ],
    )(h1s, h2s, cw1, cb1, cw2, cb2)


def kernel(x, edge_index, edge_attr, W1, b1, W2, b2, cw1, cb1, cw2, cb2):
    f32 = jnp.float32
    src = edge_index[0].astype(jnp.int32)
    dst = edge_index[1].astype(jnp.int32)
    ew = edge_attr.reshape(-1).astype(f32)
    xw = _tc_matvec(x, W1)[:, 0]
    xwp = jnp.pad(xw, (0, NPAD - N))
    par = jnp.zeros((16,), f32)
    par = par.at[2].set(W2[0, 0]).at[3].set(b1[0]).at[4].set(b2[0])
    h1s, h2s = _sc_main(src, dst, ew, xwp, par)
    return _tc_convs(h1s.reshape(1, NPAD), h2s.reshape(1, NPAD),
                     cw1, cb1, cw2, cb2)
